# Initial kernel scaffold; baseline (speedup 1.0000x reference)
import jax
import jax.numpy as jnp
from jax.experimental import pallas as pl


def kernel(edge_index, input_tensor, W1, b1, W2, b2):
    raise NotImplementedError("write your pallas kernel here")


# speedup vs baseline: 2.8152x; 2.8152x over previous
import functools

import jax
import jax.numpy as jnp
from jax import lax
from jax.experimental import pallas as pl
from jax.experimental.pallas import tpu as pltpu
from jax.experimental.pallas import tpu_sc as plsc

N_NODES = 10000
N_EDGES = 320000
D = 128
DH = D // 2

NC = 2
NS = 16
EDGES_PER_TILE = N_EDGES // NS
CHUNK = 80
N_CHUNKS = EDGES_PER_TILE // CHUNK
ROWS_PER_TILE = 640
N_PAD = NS * ROWS_PER_TILE


@functools.cache
def _make_sc_agg():
    mesh = plsc.VectorSubcoreMesh(core_axis_name="c", subcore_axis_name="s")

    @functools.partial(
        pl.kernel,
        mesh=mesh,
        out_type=jax.ShapeDtypeStruct((NC, N_PAD, DH), jnp.float32),
        scratch_types=[
            pltpu.VMEM((ROWS_PER_TILE, DH), jnp.float32),
            pltpu.VMEM((CHUNK,), jnp.int32),
            pltpu.VMEM((CHUNK,), jnp.int32),
            pltpu.VMEM((CHUNK, DH), jnp.float32),
            pltpu.VMEM_SHARED((N_PAD, DH), jnp.float32),
            pltpu.SemaphoreType.DMA,
        ],
        compiler_params=pltpu.CompilerParams(use_tc_tiling_on_sc=False),
    )
    def sc_agg(xs_hbm, src_hbm, dst_hbm, zeros_hbm, out_hbm,
               stage_v, src_v, dst_v, rows_v, agg_sh, sem):
        c = lax.axis_index("c")
        s = lax.axis_index("s")

        pltpu.sync_copy(zeros_hbm, stage_v)
        pltpu.sync_copy(stage_v, agg_sh.at[pl.ds(s * ROWS_PER_TILE, ROWS_PER_TILE)])
        plsc.subcore_barrier()

        ebase = s * EDGES_PER_TILE
        table = xs_hbm.at[c]

        def body(i, carry):
            base = pl.multiple_of(ebase + i * CHUNK, 8)
            pltpu.sync_copy(src_hbm.at[pl.ds(base, CHUNK)], src_v)
            pltpu.sync_copy(dst_hbm.at[pl.ds(base, CHUNK)], dst_v)
            pltpu.async_copy(table.at[src_v], rows_v, sem).wait()
            pltpu.sync_copy(rows_v, agg_sh.at[dst_v], add=True)
            return carry

        lax.fori_loop(0, N_CHUNKS, body, 0)
        plsc.subcore_barrier()

        pltpu.sync_copy(agg_sh.at[pl.ds(s * ROWS_PER_TILE, ROWS_PER_TILE)], stage_v)
        pltpu.sync_copy(stage_v, out_hbm.at[c, pl.ds(s * ROWS_PER_TILE, ROWS_PER_TILE)])

    return sc_agg


BLOCK = 400


def _tc_body(x_ref, a0_ref, a1_ref, w_ref, b_ref, o_ref, os_ref):
    agg = jnp.concatenate([a0_ref[0], a1_ref[0]], axis=1)
    h = x_ref[...] + agg
    y = jnp.dot(h, w_ref[...], preferred_element_type=jnp.float32) + b_ref[...]
    y = jnp.maximum(y, 0.0)
    o_ref[...] = y
    os_ref[0] = y[:, :DH]
    os_ref[1] = y[:, DH:]


def _tc_layer(x, agg, W, b):
    return pl.pallas_call(
        _tc_body,
        grid=(N_NODES // BLOCK,),
        in_specs=[
            pl.BlockSpec((BLOCK, D), lambda i: (i, 0)),
            pl.BlockSpec((1, BLOCK, DH), lambda i: (0, i, 0)),
            pl.BlockSpec((1, BLOCK, DH), lambda i: (1, i, 0)),
            pl.BlockSpec((D, D), lambda i: (0, 0)),
            pl.BlockSpec((1, D), lambda i: (0, 0)),
        ],
        out_specs=[
            pl.BlockSpec((BLOCK, D), lambda i: (i, 0)),
            pl.BlockSpec((2, BLOCK, DH), lambda i: (0, i, 0)),
        ],
        out_shape=[
            jax.ShapeDtypeStruct((N_NODES, D), jnp.float32),
            jax.ShapeDtypeStruct((NC, N_PAD, DH), jnp.float32),
        ],
    )(x, agg, agg, W, b.reshape(1, D))


def kernel(edge_index, input_tensor, W1, b1, W2, b2):
    src = edge_index[0].astype(jnp.int32)
    dst = edge_index[1].astype(jnp.int32)
    zeros = jnp.zeros((ROWS_PER_TILE, DH), jnp.float32)

    xs = jnp.zeros((NC, N_PAD, DH), jnp.float32)
    xs = xs.at[:, :N_NODES].set(
        input_tensor.reshape(N_NODES, NC, DH).transpose(1, 0, 2))

    sc_agg = _make_sc_agg()
    agg = sc_agg(xs, src, dst, zeros)
    h, hs = _tc_layer(input_tensor, agg, W1, b1)
    agg2 = sc_agg(hs, src, dst, zeros)
    out, _ = _tc_layer(h, agg2, W2, b2)
    return out


# speedup vs baseline: 8.7397x; 3.1045x over previous
import functools

import jax
import jax.numpy as jnp
from jax import lax
from jax.experimental import pallas as pl
from jax.experimental.pallas import tpu as pltpu
from jax.experimental.pallas import tpu_sc as plsc

N_NODES = 10000
N_EDGES = 320000
D = 128
DH = D // 2

NC = 2
NS = 16
EDGES_PER_TILE = N_EDGES // NS
CHUNK = 80
N_CHUNKS = EDGES_PER_TILE // CHUNK
BLK = 4
NBLK = N_CHUNKS // BLK
REM = N_CHUNKS - NBLK * BLK
ROW_A = 624
ROW_LAST = 640


@functools.cache
def _make_sc_agg():
    mesh = plsc.VectorSubcoreMesh(core_axis_name="c", subcore_axis_name="s")

    @functools.partial(
        pl.kernel,
        mesh=mesh,
        out_type=jax.ShapeDtypeStruct((NC, N_NODES, DH), jnp.float32),
        scratch_types=[
            pltpu.VMEM((N_CHUNKS, CHUNK), jnp.int32),
            pltpu.VMEM((N_CHUNKS, CHUNK), jnp.int32),
            pltpu.VMEM((2, BLK, CHUNK, DH), jnp.float32),
            pltpu.VMEM_SHARED((N_NODES, DH), jnp.float32),
            pltpu.SemaphoreType.DMA,
            pltpu.SemaphoreType.DMA,
        ],
        compiler_params=pltpu.CompilerParams(use_tc_tiling_on_sc=False),
    )
    def sc_agg(xs_hbm, src_hbm, dst_hbm, zeros_hbm, out_hbm,
               src_t, dst_t, rows_v, agg_sh, gsem, ssem):
        c = lax.axis_index("c")
        s = lax.axis_index("s")

        @pl.when(s < NS - 1)
        def _():
            pltpu.sync_copy(zeros_hbm.at[pl.ds(0, ROW_A)],
                            agg_sh.at[pl.ds(s * ROW_A, ROW_A)])

        @pl.when(s == NS - 1)
        def _():
            pltpu.sync_copy(zeros_hbm,
                            agg_sh.at[pl.ds((NS - 1) * ROW_A, ROW_LAST)])

        cbase = s * N_CHUNKS
        pltpu.sync_copy(src_hbm.at[pl.ds(cbase, N_CHUNKS)], src_t)
        pltpu.sync_copy(dst_hbm.at[pl.ds(cbase, N_CHUNKS)], dst_t)
        plsc.subcore_barrier()

        table = xs_hbm.at[c]

        def fire_gathers(b, slot):
            for k in range(BLK):
                pltpu.async_copy(
                    table.at[src_t.at[b * BLK + k]], rows_v.at[slot, k], gsem)

        def drain_gathers(slot):
            for k in range(BLK):
                pltpu.make_async_copy(
                    table.at[src_t.at[k]], rows_v.at[slot, k], gsem).wait()

        def fire_scatters(b, slot):
            for k in range(BLK):
                pltpu.async_copy(
                    rows_v.at[slot, k], agg_sh.at[dst_t.at[b * BLK + k]],
                    ssem, add=True)

        def drain_scatters(slot):
            for k in range(BLK):
                pltpu.make_async_copy(
                    rows_v.at[slot, k], agg_sh.at[dst_t.at[k]], ssem).wait()

        fire_gathers(0, 0)

        def pair_body(bb, carry):
            for par in (0, 1):
                b = 2 * bb + par
                slot, other = par, 1 - par
                drain_gathers(slot)

                @pl.when(b > 0)
                def _():
                    drain_scatters(other)

                @pl.when(b < NBLK - 1)
                def _():
                    fire_gathers(b + 1, other)

                fire_scatters(b, slot)
            return carry

        lax.fori_loop(0, NBLK // 2, pair_body, 0)
        drain_scatters(1)

        for r in range(REM):
            cidx = NBLK * BLK + r
            pltpu.async_copy(
                table.at[src_t.at[cidx]], rows_v.at[0, 0], gsem).wait()
            pltpu.sync_copy(rows_v.at[0, 0], agg_sh.at[dst_t.at[cidx]],
                            add=True)
        plsc.subcore_barrier()

        @pl.when(s < NS - 1)
        def _():
            pltpu.sync_copy(agg_sh.at[pl.ds(s * ROW_A, ROW_A)],
                            out_hbm.at[c, pl.ds(s * ROW_A, ROW_A)])

        @pl.when(s == NS - 1)
        def _():
            pltpu.sync_copy(agg_sh.at[pl.ds((NS - 1) * ROW_A, ROW_LAST)],
                            out_hbm.at[c, pl.ds((NS - 1) * ROW_A, ROW_LAST)])

    return sc_agg


BLOCK = 400


def _tc_body(x_ref, a0_ref, a1_ref, w_ref, b_ref, o_ref, os_ref):
    agg = jnp.concatenate([a0_ref[0], a1_ref[0]], axis=1)
    h = x_ref[...] + agg
    y = jnp.dot(h, w_ref[...], preferred_element_type=jnp.float32) + b_ref[...]
    y = jnp.maximum(y, 0.0)
    o_ref[...] = y
    os_ref[0] = y[:, :DH]
    os_ref[1] = y[:, DH:]


def _tc_layer(x, agg, W, b):
    return pl.pallas_call(
        _tc_body,
        grid=(N_NODES // BLOCK,),
        in_specs=[
            pl.BlockSpec((BLOCK, D), lambda i: (i, 0)),
            pl.BlockSpec((1, BLOCK, DH), lambda i: (0, i, 0)),
            pl.BlockSpec((1, BLOCK, DH), lambda i: (1, i, 0)),
            pl.BlockSpec((D, D), lambda i: (0, 0)),
            pl.BlockSpec((1, D), lambda i: (0, 0)),
        ],
        out_specs=[
            pl.BlockSpec((BLOCK, D), lambda i: (i, 0)),
            pl.BlockSpec((2, BLOCK, DH), lambda i: (0, i, 0)),
        ],
        out_shape=[
            jax.ShapeDtypeStruct((N_NODES, D), jnp.float32),
            jax.ShapeDtypeStruct((NC, N_NODES, DH), jnp.float32),
        ],
    )(x, agg, agg, W, b.reshape(1, D))


def kernel(edge_index, input_tensor, W1, b1, W2, b2):
    src = edge_index[0].astype(jnp.int32).reshape(N_EDGES // CHUNK, CHUNK)
    dst = edge_index[1].astype(jnp.int32).reshape(N_EDGES // CHUNK, CHUNK)
    zeros = jnp.zeros((ROW_LAST, DH), jnp.float32)

    xs = input_tensor.reshape(N_NODES, NC, DH).transpose(1, 0, 2)

    sc_agg = _make_sc_agg()
    agg = sc_agg(xs, src, dst, zeros)
    h, hs = _tc_layer(input_tensor, agg, W1, b1)
    agg2 = sc_agg(hs, src, dst, zeros)
    out, _ = _tc_layer(h, agg2, W2, b2)
    return out


# speedup vs baseline: 9.4463x; 1.0809x over previous
import functools

import jax
import jax.numpy as jnp
from jax import lax
from jax.experimental import pallas as pl
from jax.experimental.pallas import tpu as pltpu
from jax.experimental.pallas import tpu_sc as plsc

N_NODES = 10000
N_EDGES = 320000
D = 128
DH = D // 2

NC = 2
NS = 16
EDGES_PER_TILE = N_EDGES // NS
CHUNK = 80
N_CHUNKS = EDGES_PER_TILE // CHUNK
BLK = 4
NBLK = N_CHUNKS // BLK
REM = N_CHUNKS - NBLK * BLK
ROW_A = 624
ROW_LAST = 640


@functools.cache
def _make_sc_agg():
    mesh = plsc.VectorSubcoreMesh(core_axis_name="c", subcore_axis_name="s")

    @functools.partial(
        pl.kernel,
        mesh=mesh,
        out_type=jax.ShapeDtypeStruct((NC, N_NODES, DH), jnp.float32),
        scratch_types=[
            pltpu.VMEM((N_CHUNKS, CHUNK), jnp.int32),
            pltpu.VMEM((N_CHUNKS, CHUNK), jnp.int32),
            pltpu.VMEM((2, BLK, CHUNK, DH), jnp.float32),
            pltpu.VMEM_SHARED((N_NODES, DH), jnp.float32),
            pltpu.SemaphoreType.DMA,
            pltpu.SemaphoreType.DMA,
        ],
        compiler_params=pltpu.CompilerParams(use_tc_tiling_on_sc=False),
    )
    def sc_agg(xs_hbm, src_hbm, dst_hbm, zeros_hbm, out_hbm,
               src_t, dst_t, rows_v, agg_sh, gsem, ssem):
        c = lax.axis_index("c")
        s = lax.axis_index("s")

        @pl.when(s < NS - 1)
        def _():
            pltpu.sync_copy(zeros_hbm.at[pl.ds(0, ROW_A)],
                            agg_sh.at[pl.ds(s * ROW_A, ROW_A)])

        @pl.when(s == NS - 1)
        def _():
            pltpu.sync_copy(zeros_hbm,
                            agg_sh.at[pl.ds((NS - 1) * ROW_A, ROW_LAST)])

        cbase = s * N_CHUNKS
        pltpu.sync_copy(src_hbm.at[c, pl.ds(cbase, N_CHUNKS)], src_t)
        pltpu.sync_copy(dst_hbm.at[pl.ds(cbase, N_CHUNKS)], dst_t)
        plsc.subcore_barrier()

        table = xs_hbm

        def fire_gathers(b, slot):
            for k in range(BLK):
                pltpu.async_copy(
                    table.at[src_t.at[b * BLK + k]], rows_v.at[slot, k], gsem)

        def drain_gathers(slot):
            for k in range(BLK):
                pltpu.make_async_copy(
                    table.at[src_t.at[k]], rows_v.at[slot, k], gsem).wait()

        def fire_scatters(b, slot):
            for k in range(BLK):
                pltpu.async_copy(
                    rows_v.at[slot, k], agg_sh.at[dst_t.at[b * BLK + k]],
                    ssem, add=True)

        def drain_scatters(slot):
            for k in range(BLK):
                pltpu.make_async_copy(
                    rows_v.at[slot, k], agg_sh.at[dst_t.at[k]], ssem).wait()

        fire_gathers(0, 0)

        def pair_body(bb, carry):
            for par in (0, 1):
                b = 2 * bb + par
                slot, other = par, 1 - par
                drain_gathers(slot)

                @pl.when(b > 0)
                def _():
                    drain_scatters(other)

                @pl.when(b < NBLK - 1)
                def _():
                    fire_gathers(b + 1, other)

                fire_scatters(b, slot)
            return carry

        lax.fori_loop(0, NBLK // 2, pair_body, 0)
        drain_scatters(1)

        for r in range(REM):
            cidx = NBLK * BLK + r
            pltpu.async_copy(
                table.at[src_t.at[cidx]], rows_v.at[0, 0], gsem).wait()
            pltpu.sync_copy(rows_v.at[0, 0], agg_sh.at[dst_t.at[cidx]],
                            add=True)
        plsc.subcore_barrier()

        @pl.when(s < NS - 1)
        def _():
            pltpu.sync_copy(agg_sh.at[pl.ds(s * ROW_A, ROW_A)],
                            out_hbm.at[c, pl.ds(s * ROW_A, ROW_A)])

        @pl.when(s == NS - 1)
        def _():
            pltpu.sync_copy(agg_sh.at[pl.ds((NS - 1) * ROW_A, ROW_LAST)],
                            out_hbm.at[c, pl.ds((NS - 1) * ROW_A, ROW_LAST)])

    return sc_agg


BLOCK = 400


def _tc_body(x_ref, a0_ref, a1_ref, w_ref, b_ref, o_ref):
    agg = jnp.concatenate([a0_ref[0], a1_ref[0]], axis=1)
    h = x_ref[...] + agg
    y = jnp.dot(h, w_ref[...], preferred_element_type=jnp.float32) + b_ref[...]
    o_ref[...] = jnp.maximum(y, 0.0)


def _tc_layer(x, agg, W, b):
    return pl.pallas_call(
        _tc_body,
        grid=(N_NODES // BLOCK,),
        in_specs=[
            pl.BlockSpec((BLOCK, D), lambda i: (i, 0)),
            pl.BlockSpec((1, BLOCK, DH), lambda i: (0, i, 0)),
            pl.BlockSpec((1, BLOCK, DH), lambda i: (1, i, 0)),
            pl.BlockSpec((D, D), lambda i: (0, 0)),
            pl.BlockSpec((1, D), lambda i: (0, 0)),
        ],
        out_specs=pl.BlockSpec((BLOCK, D), lambda i: (i, 0)),
        out_shape=jax.ShapeDtypeStruct((N_NODES, D), jnp.float32),
    )(x, agg, agg, W, b.reshape(1, D))


def kernel(edge_index, input_tensor, W1, b1, W2, b2):
    src = edge_index[0].astype(jnp.int32) * 2
    src2 = jnp.stack([src, src + 1]).reshape(NC, N_EDGES // CHUNK, CHUNK)
    dst = edge_index[1].astype(jnp.int32).reshape(N_EDGES // CHUNK, CHUNK)
    zeros = jnp.zeros((ROW_LAST, DH), jnp.float32)

    sc_agg = _make_sc_agg()
    agg = sc_agg(input_tensor.reshape(NC * N_NODES, DH), src2, dst, zeros)
    h = _tc_layer(input_tensor, agg, W1, b1)
    agg2 = sc_agg(h.reshape(NC * N_NODES, DH), src2, dst, zeros)
    return _tc_layer(h, agg2, W2, b2)


# speedup vs baseline: 9.8996x; 1.0480x over previous
import functools

import jax
import jax.numpy as jnp
from jax import lax
from jax.experimental import pallas as pl
from jax.experimental.pallas import tpu as pltpu
from jax.experimental.pallas import tpu_sc as plsc

N_NODES = 10000
N_EDGES = 320000
D = 128
DH = D // 2

NC = 2
NS = 16
EDGES_PER_TILE = N_EDGES // NS
CHUNK = 80
N_CHUNKS = EDGES_PER_TILE // CHUNK
BLK = 4
NBLK = N_CHUNKS // BLK
REM = N_CHUNKS - NBLK * BLK
ROW_A = 624
ROW_LAST = 640


@functools.cache
def _make_sc_agg():
    mesh = plsc.VectorSubcoreMesh(core_axis_name="c", subcore_axis_name="s")

    @functools.partial(
        pl.kernel,
        mesh=mesh,
        out_type=jax.ShapeDtypeStruct((NC, N_NODES, DH), jnp.float32),
        scratch_types=[
            pltpu.VMEM((N_CHUNKS, CHUNK), jnp.int32),
            pltpu.VMEM((N_CHUNKS, CHUNK), jnp.int32),
            pltpu.VMEM((2, BLK, CHUNK, DH), jnp.float32),
            pltpu.VMEM_SHARED((N_NODES, DH), jnp.float32),
            pltpu.SemaphoreType.DMA,
            pltpu.SemaphoreType.DMA,
        ],
        compiler_params=pltpu.CompilerParams(use_tc_tiling_on_sc=False),
    )
    def sc_agg(xs_hbm, src_hbm, dst_hbm, zeros_hbm, out_hbm,
               src_t, dst_t, rows_v, agg_sh, gsem, ssem):
        c = lax.axis_index("c")
        s = lax.axis_index("s")

        @pl.when(s < NS - 1)
        def _():
            pltpu.sync_copy(zeros_hbm.at[pl.ds(0, ROW_A)],
                            agg_sh.at[pl.ds(s * ROW_A, ROW_A)])

        @pl.when(s == NS - 1)
        def _():
            pltpu.sync_copy(zeros_hbm,
                            agg_sh.at[pl.ds((NS - 1) * ROW_A, ROW_LAST)])

        cbase = s * N_CHUNKS
        pltpu.sync_copy(src_hbm.at[c, pl.ds(cbase, N_CHUNKS)], src_t)
        pltpu.sync_copy(dst_hbm.at[pl.ds(cbase, N_CHUNKS)], dst_t)
        plsc.subcore_barrier()

        table = xs_hbm

        def fire_gathers(b, slot):
            for k in range(BLK):
                pltpu.async_copy(
                    table.at[src_t.at[b * BLK + k]], rows_v.at[slot, k], gsem)

        def drain_gathers(slot):
            for k in range(BLK):
                pltpu.make_async_copy(
                    table.at[src_t.at[k]], rows_v.at[slot, k], gsem).wait()

        def fire_scatters(b, slot):
            for k in range(BLK):
                pltpu.async_copy(
                    rows_v.at[slot, k], agg_sh.at[dst_t.at[b * BLK + k]],
                    ssem, add=True)

        def drain_scatters(slot):
            for k in range(BLK):
                pltpu.make_async_copy(
                    rows_v.at[slot, k], agg_sh.at[dst_t.at[k]], ssem).wait()

        fire_gathers(0, 0)

        def pair_body(bb, carry):
            for par in (0, 1):
                b = 2 * bb + par
                slot, other = par, 1 - par
                drain_gathers(slot)

                @pl.when(b > 0)
                def _():
                    drain_scatters(other)

                @pl.when(b < NBLK - 1)
                def _():
                    fire_gathers(b + 1, other)

                fire_scatters(b, slot)
            return carry

        lax.fori_loop(0, NBLK // 2, pair_body, 0)
        drain_scatters(1)

        for r in range(REM):
            cidx = NBLK * BLK + r
            pltpu.async_copy(
                table.at[src_t.at[cidx]], rows_v.at[0, 0], gsem).wait()
            pltpu.sync_copy(rows_v.at[0, 0], agg_sh.at[dst_t.at[cidx]],
                            add=True)
        plsc.subcore_barrier()

        @pl.when(s < NS - 1)
        def _():
            pltpu.sync_copy(agg_sh.at[pl.ds(s * ROW_A, ROW_A)],
                            out_hbm.at[c, pl.ds(s * ROW_A, ROW_A)])

        @pl.when(s == NS - 1)
        def _():
            pltpu.sync_copy(agg_sh.at[pl.ds((NS - 1) * ROW_A, ROW_LAST)],
                            out_hbm.at[c, pl.ds((NS - 1) * ROW_A, ROW_LAST)])

    return sc_agg


BLOCK = 1000


def _tc_body(x_ref, a0_ref, a1_ref, w_ref, b_ref, o_ref):
    agg = jnp.concatenate([a0_ref[0], a1_ref[0]], axis=1)
    h = x_ref[...] + agg
    y = jnp.dot(h, w_ref[...], preferred_element_type=jnp.float32) + b_ref[...]
    o_ref[...] = jnp.maximum(y, 0.0)


def _tc_layer(x, agg, W, b):
    return pl.pallas_call(
        _tc_body,
        grid=(N_NODES // BLOCK,),
        in_specs=[
            pl.BlockSpec((BLOCK, D), lambda i: (i, 0)),
            pl.BlockSpec((1, BLOCK, DH), lambda i: (0, i, 0)),
            pl.BlockSpec((1, BLOCK, DH), lambda i: (1, i, 0)),
            pl.BlockSpec((D, D), lambda i: (0, 0)),
            pl.BlockSpec((1, D), lambda i: (0, 0)),
        ],
        out_specs=pl.BlockSpec((BLOCK, D), lambda i: (i, 0)),
        out_shape=jax.ShapeDtypeStruct((N_NODES, D), jnp.float32),
    )(x, agg, agg, W, b.reshape(1, D))


def kernel(edge_index, input_tensor, W1, b1, W2, b2):
    src = edge_index[0].astype(jnp.int32) * 2
    src2 = jnp.stack([src, src + 1]).reshape(NC, N_EDGES // CHUNK, CHUNK)
    dst = edge_index[1].astype(jnp.int32).reshape(N_EDGES // CHUNK, CHUNK)
    zeros = jnp.zeros((ROW_LAST, DH), jnp.float32)

    sc_agg = _make_sc_agg()
    agg = sc_agg(input_tensor.reshape(NC * N_NODES, DH), src2, dst, zeros)
    h = _tc_layer(input_tensor, agg, W1, b1)
    agg2 = sc_agg(h.reshape(NC * N_NODES, DH), src2, dst, zeros)
    return _tc_layer(h, agg2, W2, b2)


# speedup vs baseline: 12.1407x; 1.2264x over previous
import functools

import jax
import jax.numpy as jnp
from jax import lax
from jax.experimental import pallas as pl
from jax.experimental.pallas import tpu as pltpu
from jax.experimental.pallas import tpu_sc as plsc

N_NODES = 10000
N_EDGES = 320000
D = 128
DH = D // 2

NC = 2
NS = 16
EDGES_PER_TILE = N_EDGES // NS
CHUNK = 80
N_CHUNKS = EDGES_PER_TILE // CHUNK
BLK = 4
NBLK = N_CHUNKS // BLK
REM = N_CHUNKS - NBLK * BLK
NSLOT = 3
ROW_A = 624
ROW_LAST = 640


@functools.cache
def _make_sc_agg():
    mesh = plsc.VectorSubcoreMesh(core_axis_name="c", subcore_axis_name="s")

    @functools.partial(
        pl.kernel,
        mesh=mesh,
        out_type=jax.ShapeDtypeStruct((NC, N_NODES, DH), jnp.float32),
        scratch_types=[
            pltpu.VMEM((N_CHUNKS, CHUNK), jnp.int32),
            pltpu.VMEM((NSLOT, BLK, CHUNK), jnp.int32),
            pltpu.VMEM((NSLOT, BLK, CHUNK), jnp.int32),
            pltpu.VMEM((NSLOT, BLK, CHUNK, DH), jnp.float32),
            pltpu.VMEM_SHARED((N_NODES, DH), jnp.float32),
            pltpu.SemaphoreType.DMA,
            pltpu.SemaphoreType.DMA,
        ],
        compiler_params=pltpu.CompilerParams(use_tc_tiling_on_sc=False),
    )
    def sc_agg(xs_hbm, packed_hbm, zeros_hbm, out_hbm,
               packed_t, src_i, dst_i, rows_v, agg_sh, gsem, ssem):
        c = lax.axis_index("c")
        s = lax.axis_index("s")

        @pl.when(s < NS - 1)
        def _():
            pltpu.sync_copy(zeros_hbm.at[pl.ds(0, ROW_A)],
                            agg_sh.at[pl.ds(s * ROW_A, ROW_A)])

        @pl.when(s == NS - 1)
        def _():
            pltpu.sync_copy(zeros_hbm,
                            agg_sh.at[pl.ds((NS - 1) * ROW_A, ROW_LAST)])

        cbase = s * N_CHUNKS
        pltpu.sync_copy(packed_hbm.at[pl.ds(cbase, N_CHUNKS)], packed_t)
        plsc.subcore_barrier()

        table = xs_hbm
        mask = jnp.int32(0xFFFF)

        def unpack_chunk(cidx, slot, k):
            for j in range(CHUNK // 16):
                v = packed_t[cidx, pl.ds(j * 16, 16)]
                src_i[slot, k, pl.ds(j * 16, 16)] = ((v & mask) << 1) + c
                dst_i[slot, k, pl.ds(j * 16, 16)] = v >> 16

        def unpack_block(b, slot):
            for k in range(BLK):
                unpack_chunk(b * BLK + k, slot, k)

        def fire_gathers(slot):
            for k in range(BLK):
                pltpu.async_copy(
                    table.at[src_i.at[slot, k]], rows_v.at[slot, k], gsem)

        def drain_gathers(slot):
            for k in range(BLK):
                pltpu.make_async_copy(
                    table.at[src_i.at[slot, k]], rows_v.at[slot, k],
                    gsem).wait()

        def fire_scatters(slot):
            for k in range(BLK):
                pltpu.async_copy(
                    rows_v.at[slot, k], agg_sh.at[dst_i.at[slot, k]],
                    ssem, add=True)

        def drain_scatters(slot):
            for k in range(BLK):
                pltpu.make_async_copy(
                    rows_v.at[slot, k], agg_sh.at[dst_i.at[slot, k]],
                    ssem).wait()

        unpack_block(0, 0)
        fire_gathers(0)
        unpack_block(1, 1)
        fire_gathers(1)

        def body(b, slot, prev, nxt):
            drain_gathers(slot)
            fire_scatters(slot)

            @pl.when(b >= 1)
            def _():
                drain_scatters(prev)

            @pl.when(b + 2 < NBLK)
            def _():
                unpack_block(b + 2, nxt)
                fire_gathers(nxt)

        def triple_body(bb, carry):
            for par in (0, 1, 2):
                b = 3 * bb + par
                body(b, par, (par + 2) % 3, (par + 2) % 3)
            return carry

        lax.fori_loop(0, (NBLK - 2) // 3, triple_body, 0)
        drain_gathers((NBLK - 2) % 3)
        fire_scatters((NBLK - 2) % 3)
        drain_scatters((NBLK - 3) % 3)
        drain_gathers((NBLK - 1) % 3)
        fire_scatters((NBLK - 1) % 3)
        drain_scatters((NBLK - 2) % 3)
        drain_scatters((NBLK - 1) % 3)

        for r in range(REM):
            cidx = NBLK * BLK + r
            unpack_chunk(cidx, 2, 0)
            pltpu.async_copy(
                table.at[src_i.at[2, 0]], rows_v.at[2, 0], gsem).wait()
            pltpu.sync_copy(rows_v.at[2, 0], agg_sh.at[dst_i.at[2, 0]],
                            add=True)
        plsc.subcore_barrier()

        @pl.when(s < NS - 1)
        def _():
            pltpu.sync_copy(agg_sh.at[pl.ds(s * ROW_A, ROW_A)],
                            out_hbm.at[c, pl.ds(s * ROW_A, ROW_A)])

        @pl.when(s == NS - 1)
        def _():
            pltpu.sync_copy(agg_sh.at[pl.ds((NS - 1) * ROW_A, ROW_LAST)],
                            out_hbm.at[c, pl.ds((NS - 1) * ROW_A, ROW_LAST)])

    return sc_agg


BLOCK = 1000


def _tc_body(x_ref, a0_ref, a1_ref, w_ref, b_ref, o_ref):
    agg = jnp.concatenate([a0_ref[0], a1_ref[0]], axis=1)
    h = x_ref[...] + agg
    y = jnp.dot(h, w_ref[...], preferred_element_type=jnp.float32) + b_ref[...]
    o_ref[...] = jnp.maximum(y, 0.0)


def _tc_layer(x, agg, W, b):
    return pl.pallas_call(
        _tc_body,
        grid=(N_NODES // BLOCK,),
        in_specs=[
            pl.BlockSpec((BLOCK, D), lambda i: (i, 0)),
            pl.BlockSpec((1, BLOCK, DH), lambda i: (0, i, 0)),
            pl.BlockSpec((1, BLOCK, DH), lambda i: (1, i, 0)),
            pl.BlockSpec((D, D), lambda i: (0, 0)),
            pl.BlockSpec((1, D), lambda i: (0, 0)),
        ],
        out_specs=pl.BlockSpec((BLOCK, D), lambda i: (i, 0)),
        out_shape=jax.ShapeDtypeStruct((N_NODES, D), jnp.float32),
    )(x, agg, agg, W, b.reshape(1, D))


def kernel(edge_index, input_tensor, W1, b1, W2, b2):
    src = edge_index[0].astype(jnp.int32)
    dst = edge_index[1].astype(jnp.int32)
    packed = (src | (dst << 16)).reshape(N_EDGES // CHUNK, CHUNK)
    zeros = jnp.zeros((ROW_LAST, DH), jnp.float32)

    sc_agg = _make_sc_agg()
    agg = sc_agg(input_tensor.reshape(NC * N_NODES, DH), packed, zeros)
    h = _tc_layer(input_tensor, agg, W1, b1)
    agg2 = sc_agg(h.reshape(NC * N_NODES, DH), packed, zeros)
    return _tc_layer(h, agg2, W2, b2)


# speedup vs baseline: 12.8079x; 1.0550x over previous
import functools

import jax
import jax.numpy as jnp
from jax import lax
from jax.experimental import pallas as pl
from jax.experimental.pallas import tpu as pltpu
from jax.experimental.pallas import tpu_sc as plsc

N_NODES = 10000
N_EDGES = 320000
D = 128
DH = D // 2

NC = 2
NS = 16
EDGES_PER_TILE = N_EDGES // NS
CHUNK = 80
N_CHUNKS = EDGES_PER_TILE // CHUNK
BLK = 4
NBLK = N_CHUNKS // BLK
REM = N_CHUNKS - NBLK * BLK
NSLOT = 3
ROW_A = 624
ROW_LAST = 640


@functools.cache
def _make_sc_agg():
    mesh = plsc.VectorSubcoreMesh(core_axis_name="c", subcore_axis_name="s")

    @functools.partial(
        pl.kernel,
        mesh=mesh,
        out_type=jax.ShapeDtypeStruct((NC, N_NODES, DH), jnp.float32),
        scratch_types=[
            pltpu.VMEM((N_CHUNKS * CHUNK,), jnp.int32),
            pltpu.VMEM((NSLOT, BLK, CHUNK), jnp.int32),
            pltpu.VMEM((NSLOT, BLK, CHUNK), jnp.int32),
            pltpu.VMEM((NSLOT, BLK, CHUNK, DH), jnp.float32),
            pltpu.VMEM_SHARED((N_NODES, DH), jnp.float32),
            pltpu.SemaphoreType.DMA,
            pltpu.SemaphoreType.DMA,
        ],
        compiler_params=pltpu.CompilerParams(use_tc_tiling_on_sc=False),
    )
    def sc_agg(xs_hbm, packed_hbm, zeros_hbm, out_hbm,
               packed_t, src_i, dst_i, rows_v, agg_sh, gsem, ssem):
        c = lax.axis_index("c")
        s = lax.axis_index("s")

        @pl.when(s < NS - 1)
        def _():
            pltpu.sync_copy(zeros_hbm.at[pl.ds(0, ROW_A)],
                            agg_sh.at[pl.ds(s * ROW_A, ROW_A)])

        @pl.when(s == NS - 1)
        def _():
            pltpu.sync_copy(zeros_hbm,
                            agg_sh.at[pl.ds((NS - 1) * ROW_A, ROW_LAST)])

        ebase = s * EDGES_PER_TILE
        pltpu.sync_copy(packed_hbm.at[pl.ds(ebase, EDGES_PER_TILE)], packed_t)
        plsc.subcore_barrier()

        table = xs_hbm
        mask = jnp.int32(0xFFFF)

        def unpack_chunk(cidx, slot, k):
            for j in range(CHUNK // 16):
                v = packed_t[pl.ds(cidx * CHUNK + j * 16, 16)]
                src_i[slot, k, pl.ds(j * 16, 16)] = ((v & mask) << 1) + c
                dst_i[slot, k, pl.ds(j * 16, 16)] = v >> 16

        def unpack_block(b, slot):
            for k in range(BLK):
                unpack_chunk(b * BLK + k, slot, k)

        def fire_gathers(slot):
            for k in range(BLK):
                pltpu.async_copy(
                    table.at[src_i.at[slot, k]], rows_v.at[slot, k], gsem)

        def drain_gathers(slot):
            for k in range(BLK):
                pltpu.make_async_copy(
                    table.at[src_i.at[slot, k]], rows_v.at[slot, k],
                    gsem).wait()

        def fire_scatters(slot):
            for k in range(BLK):
                pltpu.async_copy(
                    rows_v.at[slot, k], agg_sh.at[dst_i.at[slot, k]],
                    ssem, add=True)

        def drain_scatters(slot):
            for k in range(BLK):
                pltpu.make_async_copy(
                    rows_v.at[slot, k], agg_sh.at[dst_i.at[slot, k]],
                    ssem).wait()

        unpack_block(0, 0)
        fire_gathers(0)
        unpack_block(1, 1)
        fire_gathers(1)

        def body(b, slot, prev, nxt):
            drain_gathers(slot)
            fire_scatters(slot)

            @pl.when(b >= 1)
            def _():
                drain_scatters(prev)

            @pl.when(b + 2 < NBLK)
            def _():
                unpack_block(b + 2, nxt)
                fire_gathers(nxt)

        def triple_body(bb, carry):
            for par in (0, 1, 2):
                b = 3 * bb + par
                body(b, par, (par + 2) % 3, (par + 2) % 3)
            return carry

        lax.fori_loop(0, (NBLK - 2) // 3, triple_body, 0)
        drain_gathers((NBLK - 2) % 3)
        fire_scatters((NBLK - 2) % 3)
        drain_scatters((NBLK - 3) % 3)
        drain_gathers((NBLK - 1) % 3)
        fire_scatters((NBLK - 1) % 3)
        drain_scatters((NBLK - 2) % 3)
        drain_scatters((NBLK - 1) % 3)

        for r in range(REM):
            cidx = NBLK * BLK + r
            unpack_chunk(cidx, 2, 0)
            pltpu.async_copy(
                table.at[src_i.at[2, 0]], rows_v.at[2, 0], gsem).wait()
            pltpu.sync_copy(rows_v.at[2, 0], agg_sh.at[dst_i.at[2, 0]],
                            add=True)
        plsc.subcore_barrier()

        @pl.when(s < NS - 1)
        def _():
            pltpu.sync_copy(agg_sh.at[pl.ds(s * ROW_A, ROW_A)],
                            out_hbm.at[c, pl.ds(s * ROW_A, ROW_A)])

        @pl.when(s == NS - 1)
        def _():
            pltpu.sync_copy(agg_sh.at[pl.ds((NS - 1) * ROW_A, ROW_LAST)],
                            out_hbm.at[c, pl.ds((NS - 1) * ROW_A, ROW_LAST)])

    return sc_agg


BLOCK = 1000
def _pack_body(e_ref, o_ref):
    o_ref[...] = e_ref[0] | (e_ref[1] << 16)


def _pack_idx(edge_index):
    return pl.pallas_call(
        _pack_body,
        out_shape=jax.ShapeDtypeStruct((N_EDGES,), jnp.int32),
    )(edge_index)


def _tc_body(x_ref, a0_ref, a1_ref, w_ref, b_ref, o_ref):
    agg = jnp.concatenate([a0_ref[0], a1_ref[0]], axis=1)
    h = x_ref[...] + agg
    y = jnp.dot(h, w_ref[...], preferred_element_type=jnp.float32) + b_ref[...]
    o_ref[...] = jnp.maximum(y, 0.0)


def _tc_layer(x, agg, W, b):
    return pl.pallas_call(
        _tc_body,
        grid=(N_NODES // BLOCK,),
        in_specs=[
            pl.BlockSpec((BLOCK, D), lambda i: (i, 0)),
            pl.BlockSpec((1, BLOCK, DH), lambda i: (0, i, 0)),
            pl.BlockSpec((1, BLOCK, DH), lambda i: (1, i, 0)),
            pl.BlockSpec((D, D), lambda i: (0, 0)),
            pl.BlockSpec((1, D), lambda i: (0, 0)),
        ],
        out_specs=pl.BlockSpec((BLOCK, D), lambda i: (i, 0)),
        out_shape=jax.ShapeDtypeStruct((N_NODES, D), jnp.float32),
    )(x, agg, agg, W, b.reshape(1, D))


def kernel(edge_index, input_tensor, W1, b1, W2, b2):
    packed = _pack_idx(edge_index.astype(jnp.int32))
    zeros = jnp.zeros((ROW_LAST, DH), jnp.float32)

    sc_agg = _make_sc_agg()
    agg = sc_agg(input_tensor.reshape(NC * N_NODES, DH), packed, zeros)
    h = _tc_layer(input_tensor, agg, W1, b1)
    agg2 = sc_agg(h.reshape(NC * N_NODES, DH), packed, zeros)
    return _tc_layer(h, agg2, W2, b2)


# speedup vs baseline: 13.9309x; 1.0877x over previous
import functools

import jax
import jax.numpy as jnp
from jax import lax
from jax.experimental import pallas as pl
from jax.experimental.pallas import tpu as pltpu
from jax.experimental.pallas import tpu_sc as plsc

N_NODES = 10000
N_EDGES = 320000
D = 128
DH = D // 2

NC = 2
NS = 16
EDGES_PER_TILE = N_EDGES // NS
CHUNK = 80
N_CHUNKS = EDGES_PER_TILE // CHUNK
BLK = 4
NBLK = N_CHUNKS // BLK
REM = N_CHUNKS - NBLK * BLK
NSLOT = 3
ROW_A = 624
ROW_LAST = 640


@functools.cache
def _make_sc_agg():
    mesh = plsc.VectorSubcoreMesh(core_axis_name="c", subcore_axis_name="s")

    @functools.partial(
        pl.kernel,
        mesh=mesh,
        out_type=jax.ShapeDtypeStruct((N_NODES, D), jnp.float32),
        scratch_types=[
            pltpu.VMEM((N_CHUNKS * CHUNK,), jnp.int32),
            pltpu.VMEM((NSLOT, BLK, CHUNK), jnp.int32),
            pltpu.VMEM((NSLOT, BLK, CHUNK), jnp.int32),
            pltpu.VMEM((NSLOT, BLK, CHUNK, DH), jnp.float32),
            pltpu.VMEM_SHARED((N_NODES, DH), jnp.float32),
            pltpu.SemaphoreType.DMA,
            pltpu.SemaphoreType.DMA,
        ],
        compiler_params=pltpu.CompilerParams(use_tc_tiling_on_sc=False),
    )
    def sc_agg(xs_hbm, packed_hbm, zeros_hbm, out_hbm,
               packed_t, src_i, dst_i, rows_v, agg_sh, gsem, ssem):
        c = lax.axis_index("c")
        s = lax.axis_index("s")

        @pl.when(s < NS - 1)
        def _():
            pltpu.sync_copy(zeros_hbm.at[pl.ds(0, ROW_A)],
                            agg_sh.at[pl.ds(s * ROW_A, ROW_A)])

        @pl.when(s == NS - 1)
        def _():
            pltpu.sync_copy(zeros_hbm,
                            agg_sh.at[pl.ds((NS - 1) * ROW_A, ROW_LAST)])

        ebase = s * EDGES_PER_TILE
        pltpu.sync_copy(packed_hbm.at[pl.ds(ebase, EDGES_PER_TILE)], packed_t)
        plsc.subcore_barrier()

        table = xs_hbm
        mask = jnp.int32(0xFFFF)

        def unpack_chunk(cidx, slot, k):
            for j in range(CHUNK // 16):
                v = packed_t[pl.ds(cidx * CHUNK + j * 16, 16)]
                src_i[slot, k, pl.ds(j * 16, 16)] = ((v & mask) << 1) + c
                dst_i[slot, k, pl.ds(j * 16, 16)] = v >> 16

        def unpack_block(b, slot):
            for k in range(BLK):
                unpack_chunk(b * BLK + k, slot, k)

        def fire_gathers(slot):
            for k in range(BLK):
                pltpu.async_copy(
                    table.at[src_i.at[slot, k]], rows_v.at[slot, k], gsem)

        def drain_gathers(slot):
            for k in range(BLK):
                pltpu.make_async_copy(
                    table.at[src_i.at[slot, k]], rows_v.at[slot, k],
                    gsem).wait()

        def fire_scatters(slot):
            for k in range(BLK):
                pltpu.async_copy(
                    rows_v.at[slot, k], agg_sh.at[dst_i.at[slot, k]],
                    ssem, add=True)

        def drain_scatters(slot):
            for k in range(BLK):
                pltpu.make_async_copy(
                    rows_v.at[slot, k], agg_sh.at[dst_i.at[slot, k]],
                    ssem).wait()

        unpack_block(0, 0)
        fire_gathers(0)
        unpack_block(1, 1)
        fire_gathers(1)

        def body(b, slot, prev, nxt):
            drain_gathers(slot)
            fire_scatters(slot)

            @pl.when(b >= 1)
            def _():
                drain_scatters(prev)

            @pl.when(b + 2 < NBLK)
            def _():
                unpack_block(b + 2, nxt)
                fire_gathers(nxt)

        def triple_body(bb, carry):
            for par in (0, 1, 2):
                b = 3 * bb + par
                body(b, par, (par + 2) % 3, (par + 2) % 3)
            return carry

        lax.fori_loop(0, (NBLK - 2) // 3, triple_body, 0)
        drain_gathers((NBLK - 2) % 3)
        fire_scatters((NBLK - 2) % 3)
        drain_scatters((NBLK - 3) % 3)
        drain_gathers((NBLK - 1) % 3)
        fire_scatters((NBLK - 1) % 3)
        drain_scatters((NBLK - 2) % 3)
        drain_scatters((NBLK - 1) % 3)

        for r in range(REM):
            cidx = NBLK * BLK + r
            unpack_chunk(cidx, 2, 0)
            pltpu.async_copy(
                table.at[src_i.at[2, 0]], rows_v.at[2, 0], gsem).wait()
            pltpu.sync_copy(rows_v.at[2, 0], agg_sh.at[dst_i.at[2, 0]],
                            add=True)
        plsc.subcore_barrier()

        @pl.when(s < NS - 1)
        def _():
            pltpu.sync_copy(agg_sh.at[pl.ds(s * ROW_A, ROW_A)],
                            out_hbm.at[pl.ds(s * ROW_A, ROW_A),
                                       pl.ds(c * DH, DH)])

        @pl.when(s == NS - 1)
        def _():
            pltpu.sync_copy(agg_sh.at[pl.ds((NS - 1) * ROW_A, ROW_LAST)],
                            out_hbm.at[pl.ds((NS - 1) * ROW_A, ROW_LAST),
                                       pl.ds(c * DH, DH)])

    return sc_agg


BLOCK = 1000
def _pack_body(e_ref, o_ref):
    o_ref[...] = e_ref[0] | (e_ref[1] << 16)


def _pack_idx(edge_index):
    return pl.pallas_call(
        _pack_body,
        out_shape=jax.ShapeDtypeStruct((N_EDGES,), jnp.int32),
    )(edge_index)


def _tc_body(x_ref, a_ref, w_ref, b_ref, o_ref):
    h = x_ref[...] + a_ref[...]
    y = jnp.dot(h, w_ref[...], preferred_element_type=jnp.float32) + b_ref[...]
    o_ref[...] = jnp.maximum(y, 0.0)


def _tc_layer(x, agg, W, b):
    return pl.pallas_call(
        _tc_body,
        grid=(N_NODES // BLOCK,),
        in_specs=[
            pl.BlockSpec((BLOCK, D), lambda i: (i, 0)),
            pl.BlockSpec((BLOCK, D), lambda i: (i, 0)),
            pl.BlockSpec((D, D), lambda i: (0, 0)),
            pl.BlockSpec((1, D), lambda i: (0, 0)),
        ],
        out_specs=pl.BlockSpec((BLOCK, D), lambda i: (i, 0)),
        out_shape=jax.ShapeDtypeStruct((N_NODES, D), jnp.float32),
    )(x, agg, W, b.reshape(1, D))


def kernel(edge_index, input_tensor, W1, b1, W2, b2):
    packed = _pack_idx(edge_index.astype(jnp.int32))
    zeros = jnp.zeros((ROW_LAST, DH), jnp.float32)

    sc_agg = _make_sc_agg()
    agg = sc_agg(input_tensor.reshape(NC * N_NODES, DH), packed, zeros)
    h = _tc_layer(input_tensor, agg, W1, b1)
    agg2 = sc_agg(h.reshape(NC * N_NODES, DH), packed, zeros)
    return _tc_layer(h, agg2, W2, b2)


# speedup vs baseline: 14.2853x; 1.0254x over previous
import functools

import jax
import jax.numpy as jnp
from jax import lax
from jax.experimental import pallas as pl
from jax.experimental.pallas import tpu as pltpu
from jax.experimental.pallas import tpu_sc as plsc

N_NODES = 10000
N_EDGES = 320000
D = 128
DH = D // 2

NC = 2
NS = 16
EDGES_PER_TILE = N_EDGES // NS
CHUNK = 80
N_CHUNKS = EDGES_PER_TILE // CHUNK
BLK = 4
NBLK = N_CHUNKS // BLK
REM = N_CHUNKS - NBLK * BLK
NSLOT = 3
ROW_A = 624
ROW_LAST = 640


@functools.cache
def _make_sc_agg():
    mesh = plsc.VectorSubcoreMesh(core_axis_name="c", subcore_axis_name="s")

    @functools.partial(
        pl.kernel,
        mesh=mesh,
        out_type=jax.ShapeDtypeStruct((N_NODES, D), jnp.float32),
        scratch_types=[
            pltpu.VMEM((N_CHUNKS * CHUNK,), jnp.int32),
            pltpu.VMEM((NSLOT, BLK, CHUNK), jnp.int32),
            pltpu.VMEM((NSLOT, BLK, CHUNK), jnp.int32),
            pltpu.VMEM((NSLOT, BLK, CHUNK, DH), jnp.float32),
            pltpu.VMEM_SHARED((N_NODES, DH), jnp.float32),
            pltpu.SemaphoreType.DMA,
            pltpu.SemaphoreType.DMA,
        ],
        compiler_params=pltpu.CompilerParams(use_tc_tiling_on_sc=False),
    )
    def sc_agg(xs_hbm, packed_hbm, zeros_hbm, out_hbm,
               packed_t, src_i, dst_i, rows_v, agg_sh, gsem, ssem):
        c = lax.axis_index("c")
        s = lax.axis_index("s")

        @pl.when(s < NS - 1)
        def _():
            pltpu.sync_copy(zeros_hbm.at[pl.ds(0, ROW_A)],
                            agg_sh.at[pl.ds(s * ROW_A, ROW_A)])

        @pl.when(s == NS - 1)
        def _():
            pltpu.sync_copy(zeros_hbm,
                            agg_sh.at[pl.ds((NS - 1) * ROW_A, ROW_LAST)])

        ebase = s * EDGES_PER_TILE
        pltpu.sync_copy(packed_hbm.at[pl.ds(ebase, EDGES_PER_TILE)], packed_t)
        plsc.subcore_barrier()

        table = xs_hbm
        mask = jnp.int32(0xFFFF)

        def unpack_chunk(cidx, slot, k):
            for j in range(CHUNK // 16):
                v = packed_t[pl.ds(cidx * CHUNK + j * 16, 16)]
                src_i[slot, k, pl.ds(j * 16, 16)] = ((v & mask) << 1) + c
                dst_i[slot, k, pl.ds(j * 16, 16)] = v >> 16

        def unpack_block(b, slot):
            for k in range(BLK):
                unpack_chunk(b * BLK + k, slot, k)

        def fire_gathers(slot):
            for k in range(BLK):
                pltpu.async_copy(
                    table.at[src_i.at[slot, k]], rows_v.at[slot, k], gsem)

        def drain_gathers(slot):
            for k in range(BLK):
                pltpu.make_async_copy(
                    table.at[src_i.at[slot, k]], rows_v.at[slot, k],
                    gsem).wait()

        def fire_scatters(slot):
            for k in range(BLK):
                pltpu.async_copy(
                    rows_v.at[slot, k], agg_sh.at[dst_i.at[slot, k]],
                    ssem, add=True)

        def drain_scatters(slot):
            for k in range(BLK):
                pltpu.make_async_copy(
                    rows_v.at[slot, k], agg_sh.at[dst_i.at[slot, k]],
                    ssem).wait()

        unpack_block(0, 0)
        fire_gathers(0)
        unpack_block(1, 1)
        fire_gathers(1)

        def body(b, slot, prev, nxt):
            drain_gathers(slot)
            fire_scatters(slot)

            @pl.when(b >= 1)
            def _():
                drain_scatters(prev)

            @pl.when(b + 2 < NBLK)
            def _():
                unpack_block(b + 2, nxt)
                fire_gathers(nxt)

        def triple_body(bb, carry):
            for par in (0, 1, 2):
                b = 3 * bb + par
                body(b, par, (par + 2) % 3, (par + 2) % 3)
            return carry

        lax.fori_loop(0, (NBLK - 2) // 3, triple_body, 0)
        drain_gathers((NBLK - 2) % 3)
        fire_scatters((NBLK - 2) % 3)
        drain_scatters((NBLK - 3) % 3)
        drain_gathers((NBLK - 1) % 3)
        fire_scatters((NBLK - 1) % 3)
        drain_scatters((NBLK - 2) % 3)
        drain_scatters((NBLK - 1) % 3)

        for r in range(REM):
            cidx = NBLK * BLK + r
            unpack_chunk(cidx, 2, 0)
            pltpu.async_copy(
                table.at[src_i.at[2, 0]], rows_v.at[2, 0], gsem).wait()
            pltpu.sync_copy(rows_v.at[2, 0], agg_sh.at[dst_i.at[2, 0]],
                            add=True)
        plsc.subcore_barrier()

        @pl.when(s < NS - 1)
        def _():
            pltpu.sync_copy(agg_sh.at[pl.ds(s * ROW_A, ROW_A)],
                            out_hbm.at[pl.ds(s * ROW_A, ROW_A),
                                       pl.ds(c * DH, DH)])

        @pl.when(s == NS - 1)
        def _():
            pltpu.sync_copy(agg_sh.at[pl.ds((NS - 1) * ROW_A, ROW_LAST)],
                            out_hbm.at[pl.ds((NS - 1) * ROW_A, ROW_LAST),
                                       pl.ds(c * DH, DH)])

    return sc_agg


BLOCK = 2000
def _pack_body(e_ref, o_ref):
    o_ref[...] = e_ref[0] | (e_ref[1] << 16)


def _pack_idx(edge_index):
    return pl.pallas_call(
        _pack_body,
        out_shape=jax.ShapeDtypeStruct((N_EDGES,), jnp.int32),
    )(edge_index)


def _tc_body(x_ref, a_ref, w_ref, b_ref, o_ref):
    h = x_ref[...] + a_ref[...]
    y = jnp.dot(h, w_ref[...], preferred_element_type=jnp.float32) + b_ref[...]
    o_ref[...] = jnp.maximum(y, 0.0)


def _tc_layer(x, agg, W, b):
    return pl.pallas_call(
        _tc_body,
        grid=(N_NODES // BLOCK,),
        in_specs=[
            pl.BlockSpec((BLOCK, D), lambda i: (i, 0)),
            pl.BlockSpec((BLOCK, D), lambda i: (i, 0)),
            pl.BlockSpec((D, D), lambda i: (0, 0)),
            pl.BlockSpec((1, D), lambda i: (0, 0)),
        ],
        out_specs=pl.BlockSpec((BLOCK, D), lambda i: (i, 0)),
        out_shape=jax.ShapeDtypeStruct((N_NODES, D), jnp.float32),
    )(x, agg, W, b.reshape(1, D))


def kernel(edge_index, input_tensor, W1, b1, W2, b2):
    packed = _pack_idx(edge_index.astype(jnp.int32))
    zeros = jnp.zeros((ROW_LAST, DH), jnp.float32)

    sc_agg = _make_sc_agg()
    agg = sc_agg(input_tensor.reshape(NC * N_NODES, DH), packed, zeros)
    h = _tc_layer(input_tensor, agg, W1, b1)
    agg2 = sc_agg(h.reshape(NC * N_NODES, DH), packed, zeros)
    return _tc_layer(h, agg2, W2, b2)


# speedup vs baseline: 14.5392x; 1.0178x over previous
import functools

import jax
import jax.numpy as jnp
from jax import lax
from jax.experimental import pallas as pl
from jax.experimental.pallas import tpu as pltpu
from jax.experimental.pallas import tpu_sc as plsc

N_NODES = 10000
N_EDGES = 320000
D = 128
DH = D // 2

NC = 2
NS = 16
EDGES_PER_TILE = N_EDGES // NS
CHUNK = 80
N_CHUNKS = EDGES_PER_TILE // CHUNK
BLK = 4
NBLK = N_CHUNKS // BLK
REM = N_CHUNKS - NBLK * BLK
NSLOT = 3
ROW_A = 624
ROW_LAST = 640


@functools.cache
def _make_sc_agg():
    mesh = plsc.VectorSubcoreMesh(core_axis_name="c", subcore_axis_name="s")

    @functools.partial(
        pl.kernel,
        mesh=mesh,
        out_type=jax.ShapeDtypeStruct((N_NODES, D), jnp.float32),
        scratch_types=[
            pltpu.VMEM((N_CHUNKS * CHUNK,), jnp.int32),
            pltpu.VMEM((NSLOT, BLK, CHUNK), jnp.int32),
            pltpu.VMEM((NSLOT, BLK, CHUNK), jnp.int32),
            pltpu.VMEM((NSLOT, BLK, CHUNK, DH), jnp.float32),
            pltpu.VMEM_SHARED((N_NODES, DH), jnp.float32),
            pltpu.SemaphoreType.DMA,
            pltpu.SemaphoreType.DMA,
            pltpu.SemaphoreType.DMA,
        ],
        compiler_params=pltpu.CompilerParams(use_tc_tiling_on_sc=False),
    )
    def sc_agg(xs_hbm, packed_hbm, zeros_hbm, out_hbm,
               packed_t, src_i, dst_i, rows_v, agg_sh, gsem, ssem, zsem):
        c = lax.axis_index("c")
        s = lax.axis_index("s")

        @pl.when(s < NS - 1)
        def _():
            pltpu.async_copy(zeros_hbm.at[pl.ds(0, ROW_A)],
                             agg_sh.at[pl.ds(s * ROW_A, ROW_A)], zsem)

        @pl.when(s == NS - 1)
        def _():
            pltpu.async_copy(zeros_hbm,
                             agg_sh.at[pl.ds((NS - 1) * ROW_A, ROW_LAST)],
                             zsem)

        ebase = s * EDGES_PER_TILE
        pltpu.sync_copy(packed_hbm.at[pl.ds(ebase, EDGES_PER_TILE)], packed_t)

        table = xs_hbm
        mask = jnp.int32(0xFFFF)

        def unpack_chunk(cidx, slot, k):
            for j in range(CHUNK // 16):
                v = packed_t[pl.ds(cidx * CHUNK + j * 16, 16)]
                src_i[slot, k, pl.ds(j * 16, 16)] = ((v & mask) << 1) + c
                dst_i[slot, k, pl.ds(j * 16, 16)] = v >> 16

        def unpack_block(b, slot):
            for k in range(BLK):
                unpack_chunk(b * BLK + k, slot, k)

        def fire_gathers(slot):
            for k in range(BLK):
                pltpu.async_copy(
                    table.at[src_i.at[slot, k]], rows_v.at[slot, k], gsem)

        def drain_gathers(slot):
            for k in range(BLK):
                pltpu.make_async_copy(
                    table.at[src_i.at[slot, k]], rows_v.at[slot, k],
                    gsem).wait()

        def fire_scatters(slot):
            for k in range(BLK):
                pltpu.async_copy(
                    rows_v.at[slot, k], agg_sh.at[dst_i.at[slot, k]],
                    ssem, add=True)

        def drain_scatters(slot):
            for k in range(BLK):
                pltpu.make_async_copy(
                    rows_v.at[slot, k], agg_sh.at[dst_i.at[slot, k]],
                    ssem).wait()

        unpack_block(0, 0)
        fire_gathers(0)
        unpack_block(1, 1)
        fire_gathers(1)

        @pl.when(s < NS - 1)
        def _():
            pltpu.make_async_copy(zeros_hbm.at[pl.ds(0, ROW_A)],
                                  agg_sh.at[pl.ds(s * ROW_A, ROW_A)],
                                  zsem).wait()

        @pl.when(s == NS - 1)
        def _():
            pltpu.make_async_copy(zeros_hbm,
                                  agg_sh.at[pl.ds((NS - 1) * ROW_A, ROW_LAST)],
                                  zsem).wait()

        plsc.subcore_barrier()

        def body(b, slot, prev, nxt):
            drain_gathers(slot)
            fire_scatters(slot)

            @pl.when(b >= 1)
            def _():
                drain_scatters(prev)

            @pl.when(b + 2 < NBLK)
            def _():
                unpack_block(b + 2, nxt)
                fire_gathers(nxt)

        def triple_body(bb, carry):
            for par in (0, 1, 2):
                b = 3 * bb + par
                body(b, par, (par + 2) % 3, (par + 2) % 3)
            return carry

        lax.fori_loop(0, (NBLK - 2) // 3, triple_body, 0)
        drain_gathers((NBLK - 2) % 3)
        fire_scatters((NBLK - 2) % 3)
        drain_scatters((NBLK - 3) % 3)
        drain_gathers((NBLK - 1) % 3)
        fire_scatters((NBLK - 1) % 3)
        drain_scatters((NBLK - 2) % 3)
        drain_scatters((NBLK - 1) % 3)

        for r in range(REM):
            cidx = NBLK * BLK + r
            unpack_chunk(cidx, 2, 0)
            pltpu.async_copy(
                table.at[src_i.at[2, 0]], rows_v.at[2, 0], gsem).wait()
            pltpu.sync_copy(rows_v.at[2, 0], agg_sh.at[dst_i.at[2, 0]],
                            add=True)
        plsc.subcore_barrier()

        @pl.when(s < NS - 1)
        def _():
            pltpu.sync_copy(agg_sh.at[pl.ds(s * ROW_A, ROW_A)],
                            out_hbm.at[pl.ds(s * ROW_A, ROW_A),
                                       pl.ds(c * DH, DH)])

        @pl.when(s == NS - 1)
        def _():
            pltpu.sync_copy(agg_sh.at[pl.ds((NS - 1) * ROW_A, ROW_LAST)],
                            out_hbm.at[pl.ds((NS - 1) * ROW_A, ROW_LAST),
                                       pl.ds(c * DH, DH)])

    return sc_agg


BLOCK = 2000
def _pack_body(e_ref, o_ref):
    o_ref[...] = e_ref[0] | (e_ref[1] << 16)


def _pack_idx(edge_index):
    return pl.pallas_call(
        _pack_body,
        out_shape=jax.ShapeDtypeStruct((N_EDGES,), jnp.int32),
    )(edge_index)


def _tc_body(x_ref, a_ref, w_ref, b_ref, o_ref):
    h = x_ref[...] + a_ref[...]
    y = jnp.dot(h, w_ref[...], preferred_element_type=jnp.float32) + b_ref[...]
    o_ref[...] = jnp.maximum(y, 0.0)


def _tc_layer(x, agg, W, b):
    return pl.pallas_call(
        _tc_body,
        grid=(N_NODES // BLOCK,),
        in_specs=[
            pl.BlockSpec((BLOCK, D), lambda i: (i, 0)),
            pl.BlockSpec((BLOCK, D), lambda i: (i, 0)),
            pl.BlockSpec((D, D), lambda i: (0, 0)),
            pl.BlockSpec((1, D), lambda i: (0, 0)),
        ],
        out_specs=pl.BlockSpec((BLOCK, D), lambda i: (i, 0)),
        out_shape=jax.ShapeDtypeStruct((N_NODES, D), jnp.float32),
    )(x, agg, W, b.reshape(1, D))


def kernel(edge_index, input_tensor, W1, b1, W2, b2):
    packed = _pack_idx(edge_index.astype(jnp.int32))
    zeros = jnp.zeros((ROW_LAST, DH), jnp.float32)

    sc_agg = _make_sc_agg()
    agg = sc_agg(input_tensor.reshape(NC * N_NODES, DH), packed, zeros)
    h = _tc_layer(input_tensor, agg, W1, b1)
    agg2 = sc_agg(h.reshape(NC * N_NODES, DH), packed, zeros)
    return _tc_layer(h, agg2, W2, b2)


# speedup vs baseline: 14.7768x; 1.0163x over previous
import functools

import jax
import jax.numpy as jnp
from jax import lax
from jax.experimental import pallas as pl
from jax.experimental.pallas import tpu as pltpu
from jax.experimental.pallas import tpu_sc as plsc

N_NODES = 10000
N_EDGES = 320000
D = 128
DH = D // 2

NC = 2
NS = 16
EDGES_PER_TILE = N_EDGES // NS
CHUNK = 80
N_CHUNKS = EDGES_PER_TILE // CHUNK
BLK = 4
NBLK = N_CHUNKS // BLK
REM = N_CHUNKS - NBLK * BLK
NSLOT = 3
ROW_A = 624
ROW_LAST = 640


@functools.cache
def _make_sc_agg():
    mesh = plsc.VectorSubcoreMesh(core_axis_name="c", subcore_axis_name="s")

    @functools.partial(
        pl.kernel,
        mesh=mesh,
        out_type=jax.ShapeDtypeStruct((N_NODES, D), jnp.float32),
        scratch_types=[
            pltpu.VMEM((N_CHUNKS * CHUNK,), jnp.int32),
            pltpu.VMEM((NSLOT, BLK, CHUNK), jnp.int32),
            pltpu.VMEM((NSLOT, BLK, CHUNK), jnp.int32),
            pltpu.VMEM((NSLOT, BLK, CHUNK, DH), jnp.float32),
            pltpu.VMEM_SHARED((N_NODES, DH), jnp.float32),
            pltpu.SemaphoreType.DMA,
            pltpu.SemaphoreType.DMA,
            pltpu.SemaphoreType.DMA,
        ],
        compiler_params=pltpu.CompilerParams(use_tc_tiling_on_sc=False),
    )
    def sc_agg(xs_hbm, packed_hbm, zeros_hbm, out_hbm,
               packed_t, src_i, dst_i, rows_v, agg_sh, gsem, ssem, zsem):
        c = lax.axis_index("c")
        s = lax.axis_index("s")

        @pl.when(s < NS - 1)
        def _():
            pltpu.async_copy(zeros_hbm.at[pl.ds(0, ROW_A)],
                             agg_sh.at[pl.ds(s * ROW_A, ROW_A)], zsem)

        @pl.when(s == NS - 1)
        def _():
            pltpu.async_copy(zeros_hbm,
                             agg_sh.at[pl.ds((NS - 1) * ROW_A, ROW_LAST)],
                             zsem)

        ebase = s * EDGES_PER_TILE
        pltpu.sync_copy(packed_hbm.at[pl.ds(ebase, EDGES_PER_TILE)], packed_t)

        table = xs_hbm
        mask = jnp.int32(0xFFFF)

        def unpack_chunk(cidx, slot, k):
            for j in range(CHUNK // 16):
                v = packed_t[pl.ds(cidx * CHUNK + j * 16, 16)]
                src_i[slot, k, pl.ds(j * 16, 16)] = ((v & mask) << 1) + c
                dst_i[slot, k, pl.ds(j * 16, 16)] = v >> 16

        def unpack_block(b, slot):
            for k in range(BLK):
                unpack_chunk(b * BLK + k, slot, k)

        def fire_gathers(slot, n=BLK):
            for k in range(n):
                pltpu.async_copy(
                    table.at[src_i.at[slot, k]], rows_v.at[slot, k], gsem)

        def drain_gathers(slot, n=BLK):
            for k in range(n):
                pltpu.make_async_copy(
                    table.at[src_i.at[slot, k]], rows_v.at[slot, k],
                    gsem).wait()

        def fire_scatters(slot, n=BLK):
            for k in range(n):
                pltpu.async_copy(
                    rows_v.at[slot, k], agg_sh.at[dst_i.at[slot, k]],
                    ssem, add=True)

        def drain_scatters(slot, n=BLK):
            for k in range(n):
                pltpu.make_async_copy(
                    rows_v.at[slot, k], agg_sh.at[dst_i.at[slot, k]],
                    ssem).wait()

        unpack_block(0, 0)
        fire_gathers(0)
        unpack_block(1, 1)
        fire_gathers(1)

        @pl.when(s < NS - 1)
        def _():
            pltpu.make_async_copy(zeros_hbm.at[pl.ds(0, ROW_A)],
                                  agg_sh.at[pl.ds(s * ROW_A, ROW_A)],
                                  zsem).wait()

        @pl.when(s == NS - 1)
        def _():
            pltpu.make_async_copy(zeros_hbm,
                                  agg_sh.at[pl.ds((NS - 1) * ROW_A, ROW_LAST)],
                                  zsem).wait()

        plsc.subcore_barrier()

        def body(b, slot, prev, nxt):
            drain_gathers(slot)
            fire_scatters(slot)

            @pl.when(b >= 1)
            def _():
                drain_scatters(prev)

            @pl.when(b + 2 < NBLK)
            def _():
                unpack_block(b + 2, nxt)
                fire_gathers(nxt)

        def triple_body(bb, carry):
            for par in (0, 1, 2):
                b = 3 * bb + par
                body(b, par, (par + 2) % 3, (par + 2) % 3)
            return carry

        lax.fori_loop(0, (NBLK - 2) // 3, triple_body, 0)
        drain_gathers((NBLK - 2) % 3)
        fire_scatters((NBLK - 2) % 3)
        drain_scatters((NBLK - 3) % 3)
        for r in range(REM):
            unpack_chunk(NBLK * BLK + r, 2, r)
        fire_gathers(2, REM)
        drain_gathers((NBLK - 1) % 3)
        fire_scatters((NBLK - 1) % 3)
        drain_scatters((NBLK - 2) % 3)
        drain_gathers(2, REM)
        fire_scatters(2, REM)
        drain_scatters((NBLK - 1) % 3)
        drain_scatters(2, REM)
        plsc.subcore_barrier()

        @pl.when(s < NS - 1)
        def _():
            pltpu.sync_copy(agg_sh.at[pl.ds(s * ROW_A, ROW_A)],
                            out_hbm.at[pl.ds(s * ROW_A, ROW_A),
                                       pl.ds(c * DH, DH)])

        @pl.when(s == NS - 1)
        def _():
            pltpu.sync_copy(agg_sh.at[pl.ds((NS - 1) * ROW_A, ROW_LAST)],
                            out_hbm.at[pl.ds((NS - 1) * ROW_A, ROW_LAST),
                                       pl.ds(c * DH, DH)])

    return sc_agg


BLOCK = 2000
def _pack_body(e_ref, o_ref):
    o_ref[...] = e_ref[0] | (e_ref[1] << 16)


def _pack_idx(edge_index):
    return pl.pallas_call(
        _pack_body,
        out_shape=jax.ShapeDtypeStruct((N_EDGES,), jnp.int32),
    )(edge_index)


def _tc_body(x_ref, a_ref, w_ref, b_ref, o_ref):
    h = x_ref[...] + a_ref[...]
    y = jnp.dot(h, w_ref[...], preferred_element_type=jnp.float32) + b_ref[...]
    o_ref[...] = jnp.maximum(y, 0.0)


def _tc_layer(x, agg, W, b):
    return pl.pallas_call(
        _tc_body,
        grid=(N_NODES // BLOCK,),
        in_specs=[
            pl.BlockSpec((BLOCK, D), lambda i: (i, 0)),
            pl.BlockSpec((BLOCK, D), lambda i: (i, 0)),
            pl.BlockSpec((D, D), lambda i: (0, 0)),
            pl.BlockSpec((1, D), lambda i: (0, 0)),
        ],
        out_specs=pl.BlockSpec((BLOCK, D), lambda i: (i, 0)),
        out_shape=jax.ShapeDtypeStruct((N_NODES, D), jnp.float32),
    )(x, agg, W, b.reshape(1, D))


def kernel(edge_index, input_tensor, W1, b1, W2, b2):
    packed = _pack_idx(edge_index.astype(jnp.int32))
    zeros = jnp.zeros((ROW_LAST, DH), jnp.float32)

    sc_agg = _make_sc_agg()
    agg = sc_agg(input_tensor.reshape(NC * N_NODES, DH), packed, zeros)
    h = _tc_layer(input_tensor, agg, W1, b1)
    agg2 = sc_agg(h.reshape(NC * N_NODES, DH), packed, zeros)
    return _tc_layer(h, agg2, W2, b2)
